# Initial kernel scaffold; baseline (speedup 1.0000x reference)
import jax
import jax.numpy as jnp
from jax.experimental import pallas as pl


def kernel(x, r, weight, edge_index, edge_type):
    raise NotImplementedError("write your pallas kernel here")


# speedup vs baseline: 18.9300x; 18.9300x over previous
import functools

import jax
import jax.numpy as jnp
from jax import lax
from jax.experimental import pallas as pl
from jax.experimental.pallas import tpu as pltpu
from jax.experimental.pallas import tpu_sc as plsc

N_NODES = 10000
N_EDGES = 320000
D_FEAT = 128
NC = 2
NS = 16
NW = NC * NS
C = 128
CHUNKS = -(-N_EDGES // (NW * C))
EPW = CHUNKS * C
EPAD = NW * EPW
NPAD = 10240
TPT = NPAD // NS
BLK = 1024
GRID = NPAD // BLK

_mesh = plsc.VectorSubcoreMesh(core_axis_name="c", subcore_axis_name="s")


@functools.partial(
    pl.kernel,
    mesh=_mesh,
    out_type=jax.ShapeDtypeStruct((NC, NPAD), jnp.float32),
    scratch_types=[
        pltpu.VMEM_SHARED((NPAD,), jnp.float32),
        pltpu.VMEM((C,), jnp.int32),
        pltpu.VMEM((C,), jnp.float32),
    ],
)
def _hist_kernel(row3, zer1, deg_out, deg_sh, idx_v, ones_v):
    c = lax.axis_index("c")
    s = lax.axis_index("s")
    wid = s * NC + c
    pltpu.sync_copy(zer1, deg_sh.at[pl.ds(s * TPT, TPT)])
    for i in range(C // 16):
        ones_v[pl.ds(16 * i, 16)] = jnp.ones((16,), jnp.float32)
    plsc.subcore_barrier()

    def chunk(k, carry):
        pltpu.sync_copy(row3.at[wid, k], idx_v)
        pltpu.sync_copy(ones_v, deg_sh.at[idx_v], add=True)
        return carry

    lax.fori_loop(0, CHUNKS, chunk, 0)
    plsc.subcore_barrier()
    pltpu.sync_copy(deg_sh.at[pl.ds(s * TPT, TPT)],
                    deg_out.at[c, pl.ds(s * TPT, TPT)])


@functools.partial(
    pl.kernel,
    mesh=_mesh,
    out_type=jax.ShapeDtypeStruct((NC, NPAD, D_FEAT), jnp.float32),
    scratch_types=[
        pltpu.VMEM_SHARED((NPAD, D_FEAT), jnp.float32),
        pltpu.VMEM((C,), jnp.int32),
        pltpu.VMEM((C,), jnp.int32),
        pltpu.VMEM((C, D_FEAT), jnp.float32),
    ],
)
def _agg_kernel(xs_hbm, row3, col3, zer2, part_out, agg_sh, colv, rowv, rows_v):
    c = lax.axis_index("c")
    s = lax.axis_index("s")
    wid = s * NC + c
    pltpu.sync_copy(zer2, agg_sh.at[pl.ds(s * TPT, TPT)])
    plsc.subcore_barrier()

    def chunk(k, carry):
        pltpu.sync_copy(col3.at[wid, k], colv)
        pltpu.sync_copy(row3.at[wid, k], rowv)
        pltpu.sync_copy(xs_hbm.at[colv], rows_v)
        pltpu.sync_copy(rows_v, agg_sh.at[rowv], add=True)
        return carry

    lax.fori_loop(0, CHUNKS, chunk, 0)
    plsc.subcore_barrier()
    pltpu.sync_copy(agg_sh.at[pl.ds(s * TPT, TPT)],
                    part_out.at[c, pl.ds(s * TPT, TPT)])


def _scale_body(x_ref, deg_ref, xs_ref, dinv_ref):
    d = deg_ref[0] + deg_ref[1]
    dinv = jnp.where(d > 0.0, lax.rsqrt(jnp.where(d > 0.0, d, 1.0)), 0.0)
    dinv_ref[...] = dinv
    xs_ref[...] = x_ref[...] * dinv


def _out_body(p_ref, dinv_ref, w_ref, o_ref):
    sacc = (p_ref[0] + p_ref[1]) * dinv_ref[...]
    o_ref[...] = jnp.dot(sacc, w_ref[...],
                         preferred_element_type=jnp.float32,
                         precision=lax.Precision.HIGHEST)


def kernel(x, r, weight, edge_index, edge_type):
    del r, edge_type
    row = edge_index[0].astype(jnp.int32)
    col = edge_index[1].astype(jnp.int32)
    npad = EPAD - N_EDGES
    pad_ids = jnp.arange(npad, dtype=jnp.int32)
    row_p = jnp.concatenate([row, N_NODES + pad_ids % (NPAD - N_NODES)])
    col_p = jnp.concatenate([col, pad_ids % N_NODES])
    row3 = row_p.reshape(NW, CHUNKS, C)
    col3 = col_p.reshape(NW, CHUNKS, C)
    x_pad = jnp.pad(x, ((0, NPAD - N_NODES), (0, 0)))
    zer1 = jnp.zeros((TPT,), jnp.float32)
    zer2 = jnp.zeros((TPT, D_FEAT), jnp.float32)

    deg_p = _hist_kernel(row3, zer1)
    deg3 = deg_p.reshape(NC, NPAD, 1)

    xs, dinv2 = pl.pallas_call(
        _scale_body,
        grid=(GRID,),
        in_specs=[
            pl.BlockSpec((BLK, D_FEAT), lambda i: (i, 0)),
            pl.BlockSpec((NC, BLK, 1), lambda i: (0, i, 0)),
        ],
        out_specs=[
            pl.BlockSpec((BLK, D_FEAT), lambda i: (i, 0)),
            pl.BlockSpec((BLK, 1), lambda i: (i, 0)),
        ],
        out_shape=[
            jax.ShapeDtypeStruct((NPAD, D_FEAT), jnp.float32),
            jax.ShapeDtypeStruct((NPAD, 1), jnp.float32),
        ],
    )(x_pad, deg3)

    part = _agg_kernel(xs, row3, col3, zer2)

    out = pl.pallas_call(
        _out_body,
        grid=(GRID,),
        in_specs=[
            pl.BlockSpec((NC, BLK, D_FEAT), lambda i: (0, i, 0)),
            pl.BlockSpec((BLK, 1), lambda i: (i, 0)),
            pl.BlockSpec((D_FEAT, D_FEAT), lambda i: (0, 0)),
        ],
        out_specs=pl.BlockSpec((BLK, D_FEAT), lambda i: (i, 0)),
        out_shape=jax.ShapeDtypeStruct((NPAD, D_FEAT), jnp.float32),
    )(part, dinv2, weight)

    return out[:N_NODES]


# speedup vs baseline: 28.7926x; 1.5210x over previous
import functools

import jax
import jax.numpy as jnp
from jax import lax
from jax.experimental import pallas as pl
from jax.experimental.pallas import tpu as pltpu
from jax.experimental.pallas import tpu_sc as plsc

N_NODES = 10000
N_EDGES = 320000
D_FEAT = 128
NC = 2
NS = 16
NW = NC * NS
C = 128
CHUNKS = -(-N_EDGES // (NW * C))
EPW = CHUNKS * C
EPAD = NW * EPW
NPAD = 10240
TPT = NPAD // NS
BLK = 1024
GRID = NPAD // BLK

_mesh = plsc.VectorSubcoreMesh(core_axis_name="c", subcore_axis_name="s")


@functools.partial(
    pl.kernel,
    mesh=_mesh,
    out_type=jax.ShapeDtypeStruct((NC, NPAD), jnp.float32),
    scratch_types=[
        pltpu.VMEM_SHARED((NPAD,), jnp.float32),
        pltpu.VMEM((C,), jnp.int32),
        pltpu.VMEM((C,), jnp.int32),
        pltpu.VMEM((C,), jnp.int32),
        pltpu.VMEM((C,), jnp.float32),
        pltpu.SemaphoreType.DMA,
        pltpu.SemaphoreType.DMA,
        pltpu.SemaphoreType.DMA,
        pltpu.SemaphoreType.DMA,
        pltpu.SemaphoreType.DMA,
        pltpu.SemaphoreType.DMA,
    ],
)
def _hist_kernel(row3, zer1, deg_out, deg_sh,
                 idx0, idx1, idx2, ones_v,
                 is0, is1, is2, ss0, ss1, ss2):
    c = lax.axis_index("c")
    s = lax.axis_index("s")
    wid = s * NC + c
    idx = (idx0, idx1, idx2)
    isem = (is0, is1, is2)
    ssem = (ss0, ss1, ss2)
    pltpu.sync_copy(zer1, deg_sh.at[pl.ds(s * TPT, TPT)])
    for i in range(C // 16):
        ones_v[pl.ds(16 * i, 16)] = jnp.ones((16,), jnp.float32)
    plsc.subcore_barrier()

    def idx_cp(k, b):
        return pltpu.make_async_copy(row3.at[wid, k], idx[b], isem[b])

    def sc_cp(b):
        return pltpu.make_async_copy(ones_v, deg_sh.at[idx[b]], ssem[b])

    idx_cp(0, 0).start()

    def step(k, b):
        @pl.when(k < CHUNKS)
        def _():
            @pl.when(k >= 2)
            def _():
                sc_cp((b + 1) % 3).wait()
            idx_cp(k, b).wait()
            sc_cp(b).start(add=True)
            @pl.when(k + 1 < CHUNKS)
            def _():
                idx_cp(k + 1, (b + 1) % 3).start()

    def triple(t, carry):
        k = 3 * t
        step(k, 0)
        step(k + 1, 1)
        step(k + 2, 2)
        return carry

    lax.fori_loop(0, (CHUNKS + 2) // 3, triple, 0)
    sc_cp((CHUNKS - 2) % 3).wait()
    sc_cp((CHUNKS - 1) % 3).wait()
    plsc.subcore_barrier()
    pltpu.sync_copy(deg_sh.at[pl.ds(s * TPT, TPT)],
                    deg_out.at[c, pl.ds(s * TPT, TPT)])


@functools.partial(
    pl.kernel,
    mesh=_mesh,
    out_type=jax.ShapeDtypeStruct((NC, NPAD, D_FEAT), jnp.float32),
    scratch_types=[
        pltpu.VMEM_SHARED((NPAD, D_FEAT), jnp.float32),
        pltpu.VMEM((C,), jnp.int32),
        pltpu.VMEM((C,), jnp.int32),
        pltpu.VMEM((C,), jnp.int32),
        pltpu.VMEM((C,), jnp.int32),
        pltpu.VMEM((C, D_FEAT), jnp.float32),
        pltpu.VMEM((C, D_FEAT), jnp.float32),
        pltpu.SemaphoreType.DMA,
        pltpu.SemaphoreType.DMA,
        pltpu.SemaphoreType.DMA,
        pltpu.SemaphoreType.DMA,
        pltpu.SemaphoreType.DMA,
        pltpu.SemaphoreType.DMA,
        pltpu.SemaphoreType.DMA,
        pltpu.SemaphoreType.DMA,
    ],
)
def _agg_kernel(xs_hbm, row3, col3, zer2, part_out, agg_sh,
                colv0, colv1, rowv0, rowv1, rows0, rows1,
                cs0, cs1, rs0, rs1, gs0, gs1, ss0, ss1):
    c = lax.axis_index("c")
    s = lax.axis_index("s")
    wid = s * NC + c
    colv = (colv0, colv1)
    rowv = (rowv0, rowv1)
    rows = (rows0, rows1)
    cs = (cs0, cs1)
    rs = (rs0, rs1)
    gs = (gs0, gs1)
    ss = (ss0, ss1)
    pltpu.sync_copy(zer2, agg_sh.at[pl.ds(s * TPT, TPT)])
    plsc.subcore_barrier()

    def col_cp(k, b):
        return pltpu.make_async_copy(col3.at[wid, k], colv[b], cs[b])

    def row_cp(k, b):
        return pltpu.make_async_copy(row3.at[wid, k], rowv[b], rs[b])

    def gat_cp(b):
        return pltpu.make_async_copy(xs_hbm.at[colv[b]], rows[b], gs[b])

    def sc_cp(b):
        return pltpu.make_async_copy(rows[b], agg_sh.at[rowv[b]], ss[b])

    col_cp(0, 0).start()

    def step(k, b):
        @pl.when(k < CHUNKS)
        def _():
            @pl.when(k >= 2)
            def _():
                sc_cp(b).wait()
            row_cp(k, b).start()
            col_cp(k, b).wait()
            gat_cp(b).start()
            @pl.when(k + 1 < CHUNKS)
            def _():
                col_cp(k + 1, 1 - b).start()
            gat_cp(b).wait()
            row_cp(k, b).wait()
            sc_cp(b).start(add=True)

    def pair(t, carry):
        k = 2 * t
        step(k, 0)
        step(k + 1, 1)
        return carry

    lax.fori_loop(0, (CHUNKS + 1) // 2, pair, 0)
    sc_cp((CHUNKS - 2) % 2).wait()
    sc_cp((CHUNKS - 1) % 2).wait()
    plsc.subcore_barrier()
    pltpu.sync_copy(agg_sh.at[pl.ds(s * TPT, TPT)],
                    part_out.at[c, pl.ds(s * TPT, TPT)])


def _scale_body(x_ref, deg_ref, xs_ref, dinv_ref):
    d = deg_ref[0] + deg_ref[1]
    dinv = jnp.where(d > 0.0, lax.rsqrt(jnp.where(d > 0.0, d, 1.0)), 0.0)
    dinv_ref[...] = dinv
    xs_ref[...] = x_ref[...] * dinv


def _out_body(p_ref, dinv_ref, w_ref, o_ref):
    sacc = (p_ref[0] + p_ref[1]) * dinv_ref[...]
    o_ref[...] = jnp.dot(sacc, w_ref[...],
                         preferred_element_type=jnp.float32,
                         precision=lax.Precision.HIGHEST)


def kernel(x, r, weight, edge_index, edge_type):
    del r, edge_type
    row = edge_index[0].astype(jnp.int32)
    col = edge_index[1].astype(jnp.int32)
    npad = EPAD - N_EDGES
    pad_ids = jnp.arange(npad, dtype=jnp.int32)
    row_p = jnp.concatenate([row, N_NODES + pad_ids % (NPAD - N_NODES)])
    col_p = jnp.concatenate([col, pad_ids % N_NODES])
    row3 = row_p.reshape(NW, CHUNKS, C)
    col3 = col_p.reshape(NW, CHUNKS, C)
    x_pad = jnp.pad(x, ((0, NPAD - N_NODES), (0, 0)))
    zer1 = jnp.zeros((TPT,), jnp.float32)
    zer2 = jnp.zeros((TPT, D_FEAT), jnp.float32)

    deg_p = _hist_kernel(row3, zer1)
    deg3 = deg_p.reshape(NC, NPAD, 1)

    xs, dinv2 = pl.pallas_call(
        _scale_body,
        grid=(GRID,),
        in_specs=[
            pl.BlockSpec((BLK, D_FEAT), lambda i: (i, 0)),
            pl.BlockSpec((NC, BLK, 1), lambda i: (0, i, 0)),
        ],
        out_specs=[
            pl.BlockSpec((BLK, D_FEAT), lambda i: (i, 0)),
            pl.BlockSpec((BLK, 1), lambda i: (i, 0)),
        ],
        out_shape=[
            jax.ShapeDtypeStruct((NPAD, D_FEAT), jnp.float32),
            jax.ShapeDtypeStruct((NPAD, 1), jnp.float32),
        ],
    )(x_pad, deg3)

    part = _agg_kernel(xs, row3, col3, zer2)

    out = pl.pallas_call(
        _out_body,
        grid=(GRID,),
        in_specs=[
            pl.BlockSpec((NC, BLK, D_FEAT), lambda i: (0, i, 0)),
            pl.BlockSpec((BLK, 1), lambda i: (i, 0)),
            pl.BlockSpec((D_FEAT, D_FEAT), lambda i: (0, 0)),
        ],
        out_specs=pl.BlockSpec((BLK, D_FEAT), lambda i: (i, 0)),
        out_shape=jax.ShapeDtypeStruct((NPAD, D_FEAT), jnp.float32),
    )(part, dinv2, weight)

    return out[:N_NODES]


# speedup vs baseline: 40.0066x; 1.3895x over previous
import functools

import jax
import jax.numpy as jnp
from jax import lax
from jax.experimental import pallas as pl
from jax.experimental.pallas import tpu as pltpu
from jax.experimental.pallas import tpu_sc as plsc

N_NODES = 10000
N_EDGES = 320000
D_FEAT = 128
NC = 2
NS = 16
NW = NC * NS
C = 128
CHUNKS = -(-N_EDGES // (NW * C))
EPW = CHUNKS * C
EPAD = NW * EPW
DEGR = 10240
DTPT = DEGR // NS
AGGR = 10112
ATPT = AGGR // NS
OPT = N_NODES // NS
BLK = 1000
GRID = N_NODES // BLK

_mesh = plsc.VectorSubcoreMesh(core_axis_name="c", subcore_axis_name="s")


@functools.partial(
    pl.kernel,
    mesh=_mesh,
    out_type=jax.ShapeDtypeStruct((NC, DEGR), jnp.float32),
    scratch_types=[
        pltpu.VMEM_SHARED((DEGR,), jnp.float32),
        pltpu.VMEM((CHUNKS, C), jnp.int32),
        pltpu.VMEM((C,), jnp.float32),
        pltpu.SemaphoreType.DMA,
        pltpu.SemaphoreType.DMA,
        pltpu.SemaphoreType.DMA,
        pltpu.SemaphoreType.DMA,
        pltpu.SemaphoreType.DMA,
    ],
)
def _hist_kernel(row3, zer1, deg_out, deg_sh,
                 idx_all, ones_v, il, ss0, ss1, ss2, ss3):
    c = lax.axis_index("c")
    s = lax.axis_index("s")
    wid = s * NC + c
    ssem = (ss0, ss1, ss2, ss3)
    pltpu.make_async_copy(row3.at[wid], idx_all, il).start()
    pltpu.sync_copy(zer1, deg_sh.at[pl.ds(s * DTPT, DTPT)])
    for i in range(C // 16):
        ones_v[pl.ds(16 * i, 16)] = jnp.ones((16,), jnp.float32)
    pltpu.make_async_copy(row3.at[wid], idx_all, il).wait()
    plsc.subcore_barrier()

    def sc_at(k, b):
        return pltpu.make_async_copy(ones_v, deg_sh.at[idx_all.at[k]],
                                     ssem[b])

    def step(k, b):
        @pl.when(k < CHUNKS)
        def _():
            @pl.when(k >= 4)
            def _():
                sc_at(k - 4, b).wait()
            sc_at(k, b).start(add=True)

    def quad(t, carry):
        k = 4 * t
        step(k, 0)
        step(k + 1, 1)
        step(k + 2, 2)
        step(k + 3, 3)
        return carry

    lax.fori_loop(0, (CHUNKS + 3) // 4, quad, 0)
    for j in range(CHUNKS - 4, CHUNKS):
        sc_at(j, j % 4).wait()
    plsc.subcore_barrier()
    pltpu.sync_copy(deg_sh.at[pl.ds(s * DTPT, DTPT)],
                    deg_out.at[c, pl.ds(s * DTPT, DTPT)])


@functools.partial(
    pl.kernel,
    mesh=_mesh,
    out_type=jax.ShapeDtypeStruct((NC, AGGR, D_FEAT), jnp.float32),
    scratch_types=[
        pltpu.VMEM_SHARED((AGGR, D_FEAT), jnp.float32),
        pltpu.VMEM((C,), jnp.int32),
        pltpu.VMEM((C,), jnp.int32),
        pltpu.VMEM((C,), jnp.int32),
        pltpu.VMEM((C,), jnp.int32),
        pltpu.VMEM((C,), jnp.int32),
        pltpu.VMEM((C,), jnp.int32),
        pltpu.VMEM((C, D_FEAT), jnp.float32),
        pltpu.VMEM((C, D_FEAT), jnp.float32),
        pltpu.VMEM((C, D_FEAT), jnp.float32),
        pltpu.SemaphoreType.DMA,
        pltpu.SemaphoreType.DMA,
        pltpu.SemaphoreType.DMA,
        pltpu.SemaphoreType.DMA,
        pltpu.SemaphoreType.DMA,
        pltpu.SemaphoreType.DMA,
        pltpu.SemaphoreType.DMA,
        pltpu.SemaphoreType.DMA,
        pltpu.SemaphoreType.DMA,
        pltpu.SemaphoreType.DMA,
        pltpu.SemaphoreType.DMA,
        pltpu.SemaphoreType.DMA,
    ],
)
def _agg_kernel(xs_hbm, row3, col3, zer2, part_out, agg_sh,
                colv0, colv1, colv2, rowv0, rowv1, rowv2,
                rows0, rows1, rows2,
                cs0, cs1, cs2, rs0, rs1, rs2,
                gs0, gs1, gs2, ss0, ss1, ss2):
    c = lax.axis_index("c")
    s = lax.axis_index("s")
    wid = s * NC + c
    colv = (colv0, colv1, colv2)
    rowv = (rowv0, rowv1, rowv2)
    rows = (rows0, rows1, rows2)
    cs = (cs0, cs1, cs2)
    rs = (rs0, rs1, rs2)
    gs = (gs0, gs1, gs2)
    ss = (ss0, ss1, ss2)
    pltpu.sync_copy(zer2, agg_sh.at[pl.ds(s * ATPT, ATPT)])
    plsc.subcore_barrier()

    def col_cp(k, b):
        return pltpu.make_async_copy(col3.at[wid, k], colv[b], cs[b])

    def row_cp(k, b):
        return pltpu.make_async_copy(row3.at[wid, k], rowv[b], rs[b])

    def gat_cp(b):
        return pltpu.make_async_copy(xs_hbm.at[colv[b]], rows[b], gs[b])

    def sc_cp(b):
        return pltpu.make_async_copy(rows[b], agg_sh.at[rowv[b]], ss[b])

    col_cp(0, 0).start()
    row_cp(0, 0).start()
    col_cp(1, 1).start()
    row_cp(1, 1).start()
    col_cp(0, 0).wait()
    gat_cp(0).start()

    def step(k, b):
        bn = (b + 1) % 3
        bnn = (b + 2) % 3

        @pl.when(k < CHUNKS)
        def _():
            @pl.when(k >= 2)
            def _():
                sc_cp(bn).wait()
            @pl.when(k + 1 < CHUNKS)
            def _():
                col_cp(k + 1, bn).wait()
                gat_cp(bn).start()
                @pl.when(k >= 1)
                def _():
                    row_cp(k + 1, bn).start()
            @pl.when(k + 2 < CHUNKS)
            def _():
                col_cp(k + 2, bnn).start()
            gat_cp(b).wait()
            row_cp(k, b).wait()
            sc_cp(b).start(add=True)

    def triple(t, carry):
        k = 3 * t
        step(k, 0)
        step(k + 1, 1)
        step(k + 2, 2)
        return carry

    lax.fori_loop(0, (CHUNKS + 2) // 3, triple, 0)
    sc_cp((CHUNKS - 2) % 3).wait()
    sc_cp((CHUNKS - 1) % 3).wait()
    plsc.subcore_barrier()
    pltpu.sync_copy(agg_sh.at[pl.ds(s * ATPT, ATPT)],
                    part_out.at[c, pl.ds(s * ATPT, ATPT)])


def _scale_body(x_ref, deg_ref, xs_ref, dinv_ref):
    d = deg_ref[0] + deg_ref[1]
    dinv = jnp.where(d > 0.0, lax.rsqrt(jnp.where(d > 0.0, d, 1.0)), 0.0)
    dinv_ref[...] = dinv
    xs_ref[...] = x_ref[...] * dinv


def _out_body(p_ref, dinv_ref, w_ref, o_ref):
    sacc = (p_ref[0] + p_ref[1]) * dinv_ref[...]
    o_ref[...] = jnp.dot(sacc, w_ref[...],
                         preferred_element_type=jnp.float32,
                         precision=lax.Precision.HIGHEST)


def kernel(x, r, weight, edge_index, edge_type):
    del r, edge_type
    row = edge_index[0].astype(jnp.int32)
    col = edge_index[1].astype(jnp.int32)
    npad = EPAD - N_EDGES
    pad_ids = jnp.arange(npad, dtype=jnp.int32)
    row_p = jnp.concatenate([row, N_NODES + pad_ids % (AGGR - N_NODES)])
    col_p = jnp.concatenate([col, pad_ids % N_NODES])
    row3 = row_p.reshape(NW, CHUNKS, C)
    col3 = col_p.reshape(NW, CHUNKS, C)
    zer1 = jnp.zeros((DTPT,), jnp.float32)
    zer2 = jnp.zeros((ATPT, D_FEAT), jnp.float32)

    deg_p = _hist_kernel(row3, zer1)
    deg3 = deg_p[:, :N_NODES].reshape(NC, N_NODES, 1)

    xs, dinv2 = pl.pallas_call(
        _scale_body,
        grid=(GRID,),
        in_specs=[
            pl.BlockSpec((BLK, D_FEAT), lambda i: (i, 0)),
            pl.BlockSpec((NC, BLK, 1), lambda i: (0, i, 0)),
        ],
        out_specs=[
            pl.BlockSpec((BLK, D_FEAT), lambda i: (i, 0)),
            pl.BlockSpec((BLK, 1), lambda i: (i, 0)),
        ],
        out_shape=[
            jax.ShapeDtypeStruct((N_NODES, D_FEAT), jnp.float32),
            jax.ShapeDtypeStruct((N_NODES, 1), jnp.float32),
        ],
    )(x, deg3)

    part = _agg_kernel(xs, row3, col3, zer2)

    out = pl.pallas_call(
        _out_body,
        grid=(GRID,),
        in_specs=[
            pl.BlockSpec((NC, BLK, D_FEAT), lambda i: (0, i, 0)),
            pl.BlockSpec((BLK, 1), lambda i: (i, 0)),
            pl.BlockSpec((D_FEAT, D_FEAT), lambda i: (0, 0)),
        ],
        out_specs=pl.BlockSpec((BLK, D_FEAT), lambda i: (i, 0)),
        out_shape=jax.ShapeDtypeStruct((N_NODES, D_FEAT), jnp.float32),
    )(part, dinv2, weight)

    return out


# speedup vs baseline: 41.1272x; 1.0280x over previous
import functools

import jax
import jax.numpy as jnp
from jax import lax
from jax.experimental import pallas as pl
from jax.experimental.pallas import tpu as pltpu
from jax.experimental.pallas import tpu_sc as plsc

N_NODES = 10000
N_EDGES = 320000
D_FEAT = 128
NC = 2
NS = 16
NW = NC * NS
C = 128
CHUNKS = -(-N_EDGES // (NW * C))
EPW = CHUNKS * C
EPAD = NW * EPW
DEGR = 10240
DTPT = DEGR // NS
AGGR = 10112
ATPT = AGGR // NS
OPT = N_NODES // NS
BLK = 5000
GRID = N_NODES // BLK

_mesh = plsc.VectorSubcoreMesh(core_axis_name="c", subcore_axis_name="s")


@functools.partial(
    pl.kernel,
    mesh=_mesh,
    out_type=jax.ShapeDtypeStruct((NC, DEGR), jnp.float32),
    scratch_types=[
        pltpu.VMEM_SHARED((DEGR,), jnp.float32),
        pltpu.VMEM((CHUNKS, C), jnp.int32),
        pltpu.VMEM((C,), jnp.float32),
        pltpu.SemaphoreType.DMA,
        pltpu.SemaphoreType.DMA,
        pltpu.SemaphoreType.DMA,
        pltpu.SemaphoreType.DMA,
        pltpu.SemaphoreType.DMA,
    ],
)
def _hist_kernel(row3, zer1, deg_out, deg_sh,
                 idx_all, ones_v, il, ss0, ss1, ss2, ss3):
    c = lax.axis_index("c")
    s = lax.axis_index("s")
    wid = s * NC + c
    ssem = (ss0, ss1, ss2, ss3)
    pltpu.make_async_copy(row3.at[wid], idx_all, il).start()
    pltpu.sync_copy(zer1, deg_sh.at[pl.ds(s * DTPT, DTPT)])
    for i in range(C // 16):
        ones_v[pl.ds(16 * i, 16)] = jnp.ones((16,), jnp.float32)
    pltpu.make_async_copy(row3.at[wid], idx_all, il).wait()
    plsc.subcore_barrier()

    def sc_at(k, b):
        return pltpu.make_async_copy(ones_v, deg_sh.at[idx_all.at[k]],
                                     ssem[b])

    def step(k, b):
        @pl.when(k < CHUNKS)
        def _():
            @pl.when(k >= 4)
            def _():
                sc_at(k - 4, b).wait()
            sc_at(k, b).start(add=True)

    def quad(t, carry):
        k = 4 * t
        step(k, 0)
        step(k + 1, 1)
        step(k + 2, 2)
        step(k + 3, 3)
        return carry

    lax.fori_loop(0, (CHUNKS + 3) // 4, quad, 0)
    for j in range(CHUNKS - 4, CHUNKS):
        sc_at(j, j % 4).wait()
    plsc.subcore_barrier()
    pltpu.sync_copy(deg_sh.at[pl.ds(s * DTPT, DTPT)],
                    deg_out.at[c, pl.ds(s * DTPT, DTPT)])


@functools.partial(
    pl.kernel,
    mesh=_mesh,
    out_type=jax.ShapeDtypeStruct((NC, AGGR, D_FEAT), jnp.float32),
    scratch_types=[
        pltpu.VMEM_SHARED((AGGR, D_FEAT), jnp.float32),
        pltpu.VMEM((C,), jnp.int32),
        pltpu.VMEM((C,), jnp.int32),
        pltpu.VMEM((C,), jnp.int32),
        pltpu.VMEM((C,), jnp.int32),
        pltpu.VMEM((C,), jnp.int32),
        pltpu.VMEM((C,), jnp.int32),
        pltpu.VMEM((C, D_FEAT), jnp.float32),
        pltpu.VMEM((C, D_FEAT), jnp.float32),
        pltpu.VMEM((C, D_FEAT), jnp.float32),
        pltpu.SemaphoreType.DMA,
        pltpu.SemaphoreType.DMA,
        pltpu.SemaphoreType.DMA,
        pltpu.SemaphoreType.DMA,
        pltpu.SemaphoreType.DMA,
        pltpu.SemaphoreType.DMA,
        pltpu.SemaphoreType.DMA,
        pltpu.SemaphoreType.DMA,
        pltpu.SemaphoreType.DMA,
        pltpu.SemaphoreType.DMA,
        pltpu.SemaphoreType.DMA,
        pltpu.SemaphoreType.DMA,
    ],
)
def _agg_kernel(xs_hbm, row3, col3, zer2, part_out, agg_sh,
                colv0, colv1, colv2, rowv0, rowv1, rowv2,
                rows0, rows1, rows2,
                cs0, cs1, cs2, rs0, rs1, rs2,
                gs0, gs1, gs2, ss0, ss1, ss2):
    c = lax.axis_index("c")
    s = lax.axis_index("s")
    wid = s * NC + c
    colv = (colv0, colv1, colv2)
    rowv = (rowv0, rowv1, rowv2)
    rows = (rows0, rows1, rows2)
    cs = (cs0, cs1, cs2)
    rs = (rs0, rs1, rs2)
    gs = (gs0, gs1, gs2)
    ss = (ss0, ss1, ss2)
    pltpu.sync_copy(zer2, agg_sh.at[pl.ds(s * ATPT, ATPT)])
    plsc.subcore_barrier()

    def col_cp(k, b):
        return pltpu.make_async_copy(col3.at[wid, k], colv[b], cs[b])

    def row_cp(k, b):
        return pltpu.make_async_copy(row3.at[wid, k], rowv[b], rs[b])

    def gat_cp(b):
        return pltpu.make_async_copy(xs_hbm.at[colv[b]], rows[b], gs[b])

    def sc_cp(b):
        return pltpu.make_async_copy(rows[b], agg_sh.at[rowv[b]], ss[b])

    col_cp(0, 0).start()
    row_cp(0, 0).start()
    col_cp(1, 1).start()
    row_cp(1, 1).start()
    col_cp(0, 0).wait()
    gat_cp(0).start()

    def step(k, b):
        bn = (b + 1) % 3
        bnn = (b + 2) % 3

        @pl.when(k < CHUNKS)
        def _():
            @pl.when(k >= 2)
            def _():
                sc_cp(bn).wait()
            @pl.when(k + 1 < CHUNKS)
            def _():
                col_cp(k + 1, bn).wait()
                gat_cp(bn).start()
                @pl.when(k >= 1)
                def _():
                    row_cp(k + 1, bn).start()
            @pl.when(k + 2 < CHUNKS)
            def _():
                col_cp(k + 2, bnn).start()
            gat_cp(b).wait()
            row_cp(k, b).wait()
            sc_cp(b).start(add=True)

    def triple(t, carry):
        k = 3 * t
        step(k, 0)
        step(k + 1, 1)
        step(k + 2, 2)
        return carry

    lax.fori_loop(0, (CHUNKS + 2) // 3, triple, 0)
    sc_cp((CHUNKS - 2) % 3).wait()
    sc_cp((CHUNKS - 1) % 3).wait()
    plsc.subcore_barrier()
    pltpu.sync_copy(agg_sh.at[pl.ds(s * ATPT, ATPT)],
                    part_out.at[c, pl.ds(s * ATPT, ATPT)])


def _scale_body(x_ref, deg_ref, xs_ref, dinv_ref):
    d = deg_ref[0] + deg_ref[1]
    dinv = jnp.where(d > 0.0, lax.rsqrt(jnp.where(d > 0.0, d, 1.0)), 0.0)
    dinv_ref[...] = dinv
    xs_ref[...] = x_ref[...] * dinv


def _out_body(p_ref, dinv_ref, w_ref, o_ref):
    sacc = (p_ref[0] + p_ref[1]) * dinv_ref[...]
    o_ref[...] = jnp.dot(sacc, w_ref[...],
                         preferred_element_type=jnp.float32,
                         precision=lax.Precision.HIGHEST)


def kernel(x, r, weight, edge_index, edge_type):
    del r, edge_type
    row = edge_index[0].astype(jnp.int32)
    col = edge_index[1].astype(jnp.int32)
    npad = EPAD - N_EDGES
    pad_ids = jnp.arange(npad, dtype=jnp.int32)
    row_p = jnp.concatenate([row, N_NODES + pad_ids % (AGGR - N_NODES)])
    col_p = jnp.concatenate([col, pad_ids % N_NODES])
    row3 = row_p.reshape(NW, CHUNKS, C)
    col3 = col_p.reshape(NW, CHUNKS, C)
    zer1 = jnp.zeros((DTPT,), jnp.float32)
    zer2 = jnp.zeros((ATPT, D_FEAT), jnp.float32)

    deg_p = _hist_kernel(row3, zer1)
    deg3 = deg_p.reshape(NC, DEGR, 1)

    xs, dinv2 = pl.pallas_call(
        _scale_body,
        grid=(GRID,),
        in_specs=[
            pl.BlockSpec((BLK, D_FEAT), lambda i: (i, 0)),
            pl.BlockSpec((NC, BLK, 1), lambda i: (0, i, 0)),
        ],
        out_specs=[
            pl.BlockSpec((BLK, D_FEAT), lambda i: (i, 0)),
            pl.BlockSpec((BLK, 1), lambda i: (i, 0)),
        ],
        out_shape=[
            jax.ShapeDtypeStruct((N_NODES, D_FEAT), jnp.float32),
            jax.ShapeDtypeStruct((N_NODES, 1), jnp.float32),
        ],
    )(x, deg3)

    part = _agg_kernel(xs, row3, col3, zer2)

    out = pl.pallas_call(
        _out_body,
        grid=(GRID,),
        in_specs=[
            pl.BlockSpec((NC, BLK, D_FEAT), lambda i: (0, i, 0)),
            pl.BlockSpec((BLK, 1), lambda i: (i, 0)),
            pl.BlockSpec((D_FEAT, D_FEAT), lambda i: (0, 0)),
        ],
        out_specs=pl.BlockSpec((BLK, D_FEAT), lambda i: (i, 0)),
        out_shape=jax.ShapeDtypeStruct((N_NODES, D_FEAT), jnp.float32),
    )(part, dinv2, weight)

    return out


# speedup vs baseline: 41.2818x; 1.0038x over previous
import functools

import jax
import jax.numpy as jnp
from jax import lax
from jax.experimental import pallas as pl
from jax.experimental.pallas import tpu as pltpu
from jax.experimental.pallas import tpu_sc as plsc

N_NODES = 10000
N_EDGES = 320000
D_FEAT = 128
NC = 2
NS = 16
NW = NC * NS
C = 128
CHUNKS = -(-N_EDGES // (NW * C))
EPW = CHUNKS * C
EPAD = NW * EPW
DEGR = 10240
DTPT = DEGR // NS
AGGR = 10112
ATPT = AGGR // NS
OPT = N_NODES // NS
BLK = 2000
GRID = N_NODES // BLK

_mesh = plsc.VectorSubcoreMesh(core_axis_name="c", subcore_axis_name="s")


@functools.partial(
    pl.kernel,
    mesh=_mesh,
    out_type=jax.ShapeDtypeStruct((NC, DEGR), jnp.float32),
    scratch_types=[
        pltpu.VMEM_SHARED((DEGR,), jnp.float32),
        pltpu.VMEM((CHUNKS, C), jnp.int32),
        pltpu.VMEM((C,), jnp.float32),
        pltpu.SemaphoreType.DMA,
        pltpu.SemaphoreType.DMA,
        pltpu.SemaphoreType.DMA,
        pltpu.SemaphoreType.DMA,
        pltpu.SemaphoreType.DMA,
    ],
)
def _hist_kernel(row3, zer1, deg_out, deg_sh,
                 idx_all, ones_v, il, ss0, ss1, ss2, ss3):
    c = lax.axis_index("c")
    s = lax.axis_index("s")
    wid = s * NC + c
    ssem = (ss0, ss1, ss2, ss3)
    pltpu.make_async_copy(row3.at[wid], idx_all, il).start()
    pltpu.sync_copy(zer1, deg_sh.at[pl.ds(s * DTPT, DTPT)])
    for i in range(C // 16):
        ones_v[pl.ds(16 * i, 16)] = jnp.ones((16,), jnp.float32)
    pltpu.make_async_copy(row3.at[wid], idx_all, il).wait()
    plsc.subcore_barrier()

    def sc_at(k, b):
        return pltpu.make_async_copy(ones_v, deg_sh.at[idx_all.at[k]],
                                     ssem[b])

    def step(k, b):
        @pl.when(k < CHUNKS)
        def _():
            @pl.when(k >= 4)
            def _():
                sc_at(k - 4, b).wait()
            sc_at(k, b).start(add=True)

    def quad(t, carry):
        k = 4 * t
        step(k, 0)
        step(k + 1, 1)
        step(k + 2, 2)
        step(k + 3, 3)
        return carry

    lax.fori_loop(0, (CHUNKS + 3) // 4, quad, 0)
    for j in range(CHUNKS - 4, CHUNKS):
        sc_at(j, j % 4).wait()
    plsc.subcore_barrier()
    pltpu.sync_copy(deg_sh.at[pl.ds(s * DTPT, DTPT)],
                    deg_out.at[c, pl.ds(s * DTPT, DTPT)])


@functools.partial(
    pl.kernel,
    mesh=_mesh,
    out_type=jax.ShapeDtypeStruct((NC, AGGR, D_FEAT), jnp.float32),
    scratch_types=[
        pltpu.VMEM_SHARED((AGGR, D_FEAT), jnp.float32),
        pltpu.VMEM((C,), jnp.int32),
        pltpu.VMEM((C,), jnp.int32),
        pltpu.VMEM((C,), jnp.int32),
        pltpu.VMEM((C,), jnp.int32),
        pltpu.VMEM((C,), jnp.int32),
        pltpu.VMEM((C,), jnp.int32),
        pltpu.VMEM((C, D_FEAT), jnp.float32),
        pltpu.VMEM((C, D_FEAT), jnp.float32),
        pltpu.VMEM((C, D_FEAT), jnp.float32),
        pltpu.SemaphoreType.DMA,
        pltpu.SemaphoreType.DMA,
        pltpu.SemaphoreType.DMA,
        pltpu.SemaphoreType.DMA,
        pltpu.SemaphoreType.DMA,
        pltpu.SemaphoreType.DMA,
        pltpu.SemaphoreType.DMA,
        pltpu.SemaphoreType.DMA,
        pltpu.SemaphoreType.DMA,
        pltpu.SemaphoreType.DMA,
        pltpu.SemaphoreType.DMA,
        pltpu.SemaphoreType.DMA,
    ],
)
def _agg_kernel(xs_hbm, row3, col3, zer2, part_out, agg_sh,
                colv0, colv1, colv2, rowv0, rowv1, rowv2,
                rows0, rows1, rows2,
                cs0, cs1, cs2, rs0, rs1, rs2,
                gs0, gs1, gs2, ss0, ss1, ss2):
    c = lax.axis_index("c")
    s = lax.axis_index("s")
    wid = s * NC + c
    colv = (colv0, colv1, colv2)
    rowv = (rowv0, rowv1, rowv2)
    rows = (rows0, rows1, rows2)
    cs = (cs0, cs1, cs2)
    rs = (rs0, rs1, rs2)
    gs = (gs0, gs1, gs2)
    ss = (ss0, ss1, ss2)
    pltpu.sync_copy(zer2, agg_sh.at[pl.ds(s * ATPT, ATPT)])
    plsc.subcore_barrier()

    def col_cp(k, b):
        return pltpu.make_async_copy(col3.at[wid, k], colv[b], cs[b])

    def row_cp(k, b):
        return pltpu.make_async_copy(row3.at[wid, k], rowv[b], rs[b])

    def gat_cp(b):
        return pltpu.make_async_copy(xs_hbm.at[colv[b]], rows[b], gs[b])

    def sc_cp(b):
        return pltpu.make_async_copy(rows[b], agg_sh.at[rowv[b]], ss[b])

    col_cp(0, 0).start()
    row_cp(0, 0).start()
    col_cp(1, 1).start()
    row_cp(1, 1).start()
    col_cp(0, 0).wait()
    gat_cp(0).start()

    def step(k, b):
        bn = (b + 1) % 3
        bnn = (b + 2) % 3

        @pl.when(k < CHUNKS)
        def _():
            @pl.when(k >= 2)
            def _():
                sc_cp(bn).wait()
            @pl.when(k + 1 < CHUNKS)
            def _():
                col_cp(k + 1, bn).wait()
                gat_cp(bn).start()
                @pl.when(k >= 1)
                def _():
                    row_cp(k + 1, bn).start()
            @pl.when(k + 2 < CHUNKS)
            def _():
                col_cp(k + 2, bnn).start()
            gat_cp(b).wait()
            row_cp(k, b).wait()
            sc_cp(b).start(add=True)

    def triple(t, carry):
        k = 3 * t
        step(k, 0)
        step(k + 1, 1)
        step(k + 2, 2)
        return carry

    lax.fori_loop(0, (CHUNKS + 2) // 3, triple, 0)
    sc_cp((CHUNKS - 2) % 3).wait()
    sc_cp((CHUNKS - 1) % 3).wait()
    plsc.subcore_barrier()
    pltpu.sync_copy(agg_sh.at[pl.ds(s * ATPT, ATPT)],
                    part_out.at[c, pl.ds(s * ATPT, ATPT)])


def _scale_body(x_ref, deg_ref, xs_ref, dinv_ref):
    d = deg_ref[0] + deg_ref[1]
    dinv = jnp.where(d > 0.0, lax.rsqrt(jnp.where(d > 0.0, d, 1.0)), 0.0)
    dinv_ref[...] = dinv
    xs_ref[...] = x_ref[...] * dinv


def _out_body(p_ref, dinv_ref, w_ref, o_ref):
    sacc = (p_ref[0] + p_ref[1]) * dinv_ref[...]
    o_ref[...] = jnp.dot(sacc, w_ref[...],
                         preferred_element_type=jnp.float32,
                         precision=lax.Precision.HIGHEST)


def kernel(x, r, weight, edge_index, edge_type):
    del r, edge_type
    row = edge_index[0].astype(jnp.int32)
    col = edge_index[1].astype(jnp.int32)
    npad = EPAD - N_EDGES
    pad_ids = jnp.arange(npad, dtype=jnp.int32)
    row_p = jnp.concatenate([row, N_NODES + pad_ids % (AGGR - N_NODES)])
    col_p = jnp.concatenate([col, pad_ids % N_NODES])
    row3 = row_p.reshape(NW, CHUNKS, C)
    col3 = col_p.reshape(NW, CHUNKS, C)
    zer1 = jnp.zeros((DTPT,), jnp.float32)
    zer2 = jnp.zeros((ATPT, D_FEAT), jnp.float32)

    deg_p = _hist_kernel(row3, zer1)
    deg3 = deg_p.reshape(NC, DEGR, 1)

    xs, dinv2 = pl.pallas_call(
        _scale_body,
        grid=(GRID,),
        in_specs=[
            pl.BlockSpec((BLK, D_FEAT), lambda i: (i, 0)),
            pl.BlockSpec((NC, BLK, 1), lambda i: (0, i, 0)),
        ],
        out_specs=[
            pl.BlockSpec((BLK, D_FEAT), lambda i: (i, 0)),
            pl.BlockSpec((BLK, 1), lambda i: (i, 0)),
        ],
        out_shape=[
            jax.ShapeDtypeStruct((N_NODES, D_FEAT), jnp.float32),
            jax.ShapeDtypeStruct((N_NODES, 1), jnp.float32),
        ],
    )(x, deg3)

    part = _agg_kernel(xs, row3, col3, zer2)

    out = pl.pallas_call(
        _out_body,
        grid=(GRID,),
        in_specs=[
            pl.BlockSpec((NC, BLK, D_FEAT), lambda i: (0, i, 0)),
            pl.BlockSpec((BLK, 1), lambda i: (i, 0)),
            pl.BlockSpec((D_FEAT, D_FEAT), lambda i: (0, 0)),
        ],
        out_specs=pl.BlockSpec((BLK, D_FEAT), lambda i: (i, 0)),
        out_shape=jax.ShapeDtypeStruct((N_NODES, D_FEAT), jnp.float32),
    )(part, dinv2, weight)

    return out
